# baseline (speedup 1.0000x reference)
import functools

import jax
import jax.numpy as jnp
from jax import lax
from jax.experimental import pallas as pl
from jax.experimental.pallas import tpu as pltpu
from jax.experimental.pallas import tpu_sc as plsc

LANES = 16
NBUF = 2


@functools.lru_cache(maxsize=None)
def _make_gather_kernel(B, S, D, V):
    info = plsc.get_sparse_core_info()
    NC, NS = info.num_cores, info.num_subcores
    NW = NC * NS
    BB = B // 128
    nq = S * BB
    assert B % 128 == 0 and D % LANES == 0 and nq % (NW * NBUF) == 0
    niter = nq // (NW * NBUF)
    mesh = plsc.VectorSubcoreMesh(core_axis_name="c", subcore_axis_name="s")

    @functools.partial(
        pl.kernel,
        mesh=mesh,
        compiler_params=pltpu.CompilerParams(use_tc_tiling_on_sc=False),
        out_type=jax.ShapeDtypeStruct((B * S, D), jnp.float32),
        scratch_types=[
            pltpu.VMEM((S, D), jnp.float32),
            [pltpu.VMEM((128,), jnp.int32)] * NBUF,
            [pltpu.VMEM((128, D), jnp.float32)] * NBUF,
            [pltpu.SemaphoreType.DMA] * NBUF,
            [pltpu.SemaphoreType.DMA] * NBUF,
            [pltpu.SemaphoreType.DMA] * NBUF,
        ],
    )
    def gather_kernel(idxf_hbm, wte_hbm, wpe_hbm, mid_hbm,
                      wpe_v, idx_v, rows_v, i_sems, g_sems, o_sems):
        wid = lax.axis_index("s") * NC + lax.axis_index("c")
        pltpu.sync_copy(wpe_hbm, wpe_v)

        def start_idx(q, b):
            return pltpu.async_copy(
                idxf_hbm.at[pl.ds(q * 128, 128)], idx_v[b], i_sems[b])

        def wait_idx(b):
            pltpu.make_async_copy(
                idxf_hbm.at[pl.ds(0, 128)], idx_v[b], i_sems[b]).wait()

        def start_gather(b):
            return pltpu.async_copy(
                wte_hbm.at[idx_v[b]], rows_v[b], g_sems[b])

        def wait_gather(b):
            pltpu.make_async_copy(
                wte_hbm.at[idx_v[b]], rows_v[b], g_sems[b]).wait()

        def wait_out(b):
            pltpu.make_async_copy(
                rows_v[b], mid_hbm.at[pl.ds(0, 128)], o_sems[b]).wait()

        for b in range(NBUF):
            start_idx(wid * NBUF + b, b)
        for b in range(NBUF):
            wait_idx(b)
            start_gather(b)

        def iter_body(i, _):
            q0 = i * NW * NBUF + wid * NBUF
            for b in range(NBUF):
                q = q0 + b
                s = q // BB
                wait_gather(b)

                @pl.when(i + 1 < niter)
                def _():
                    start_idx(q + NW * NBUF, b)

                @pl.when(i > 0)
                def _():
                    wait_out(b)

                @plsc.parallel_loop(0, 128, 2, unroll=4)
                def _(r):
                    for k in range(2):
                        for j in range(D // LANES):
                            sl = pl.ds(j * LANES, LANES)
                            plsc.addupdate(
                                rows_v[b].at[r + k, sl], wpe_v[s, sl])

                pltpu.async_copy(
                    rows_v[b], mid_hbm.at[pl.ds(q * 128, 128)], o_sems[b])

                @pl.when(i + 1 < niter)
                def _():
                    wait_idx(b)
                    start_gather(b)

            return 0

        lax.fori_loop(0, niter, iter_body, 0)
        for b in range(NBUF):
            wait_out(b)

    return gather_kernel


@functools.lru_cache(maxsize=None)
def _make_transpose_kernel(B, S, D):
    BB = B // 128
    DG = D // 8

    def body(mid_ref, out_ref):
        x = mid_ref[...]
        y = jnp.swapaxes(x, 0, 1)
        out_ref[0, :, 0, :, :] = y.reshape(DG, 8, 128)

    return pl.pallas_call(
        body,
        grid=(S, BB),
        in_specs=[pl.BlockSpec((128, D), lambda s, bg: (s * BB + bg, 0))],
        out_specs=pl.BlockSpec(
            (1, DG, 1, 8, 128), lambda s, bg: (s, 0, bg, 0, 0)),
        out_shape=jax.ShapeDtypeStruct((S, DG, BB, 8, 128), jnp.float32),
    )


def kernel(idx, wte, wpe):
    B, S = idx.shape
    V, D = wte.shape
    mid = _make_gather_kernel(B, S, D, V)(idx.T.reshape(-1), wte, wpe)
    out4 = _make_transpose_kernel(B, S, D)(mid)
    return out4.transpose(2, 4, 0, 1, 3).reshape(B, S, D)
